# initial kernel scaffold (unmeasured)
import jax
import jax.numpy as jnp
from jax import lax
from jax.experimental import pallas as pl
from jax.experimental.pallas import tpu as pltpu

EPS = 1e-5


def kernel(x, gamma, beta):
    m, n = x.shape
    n_global = 2.0 * n

    gamma2 = gamma.reshape(1, n)
    beta2 = beta.reshape(1, n)

    def body(x_ref, g_ref, b_ref, out_ref,
             local_stats, remote_stats, send_sem, recv_sem):
        my_x = lax.axis_index("x")
        my_y = lax.axis_index("y")
        peer = (my_x, 1 - my_y)

        barrier_sem = pltpu.get_barrier_semaphore()
        pl.semaphore_signal(
            barrier_sem, inc=1,
            device_id=peer, device_id_type=pl.DeviceIdType.MESH,
        )
        pl.semaphore_wait(barrier_sem, 1)

        xv = x_ref[:, :]
        local_stats[:, 0:1] = jnp.sum(xv, axis=1, keepdims=True)
        local_stats[:, 1:2] = jnp.sum(xv * xv, axis=1, keepdims=True)

        rdma = pltpu.make_async_remote_copy(
            src_ref=local_stats,
            dst_ref=remote_stats,
            send_sem=send_sem,
            recv_sem=recv_sem,
            device_id=peer,
            device_id_type=pl.DeviceIdType.MESH,
        )
        rdma.start()
        rdma.wait()

        total = local_stats[:, 0:1] + remote_stats[:, 0:1]
        total_sq = local_stats[:, 1:2] + remote_stats[:, 1:2]
        mean = total / n_global
        var = total_sq / n_global - mean * mean
        inv = lax.rsqrt(var + EPS)
        out_ref[:, :] = g_ref[0:1, :] * ((xv - mean) * inv) + b_ref[0:1, :]

    return pl.pallas_call(
        body,
        out_shape=jax.ShapeDtypeStruct((m, n), x.dtype),
        in_specs=[
            pl.BlockSpec(memory_space=pltpu.VMEM),
            pl.BlockSpec(memory_space=pltpu.VMEM),
            pl.BlockSpec(memory_space=pltpu.VMEM),
        ],
        out_specs=pl.BlockSpec(memory_space=pltpu.VMEM),
        scratch_shapes=[
            pltpu.VMEM((m, 2), jnp.float32),
            pltpu.VMEM((m, 2), jnp.float32),
            pltpu.SemaphoreType.DMA,
            pltpu.SemaphoreType.DMA,
        ],
        compiler_params=pltpu.CompilerParams(collective_id=0),
    )(x, gamma2, beta2)


# baseline (device time: 56251 ns/iter reference)
import jax
import jax.numpy as jnp
from jax import lax
from jax.experimental import pallas as pl
from jax.experimental.pallas import tpu as pltpu

EPS = 1e-5


def kernel(x, gamma, beta):
    m, n = x.shape
    n_global = 2.0 * n

    gamma2 = gamma.reshape(1, n)
    beta2 = beta.reshape(1, n)

    def body(x_ref, g_ref, b_ref, out_ref,
             local_stats, remote_stats, send_sem, recv_sem):
        my_x = lax.axis_index("x")
        my_y = lax.axis_index("y")
        peer = (my_x, 1 - my_y)

        barrier_sem = pltpu.get_barrier_semaphore()
        pl.semaphore_signal(
            barrier_sem, inc=1,
            device_id=peer, device_id_type=pl.DeviceIdType.MESH,
        )
        pl.semaphore_wait(barrier_sem, 1)

        xv = x_ref[:, :]
        local_stats[:, 0:1] = jnp.sum(xv, axis=1, keepdims=True)
        local_stats[:, 1:2] = jnp.sum(xv * xv, axis=1, keepdims=True)

        rdma = pltpu.make_async_remote_copy(
            src_ref=local_stats,
            dst_ref=remote_stats,
            send_sem=send_sem,
            recv_sem=recv_sem,
            device_id=peer,
            device_id_type=pl.DeviceIdType.MESH,
        )
        rdma.start()
        rdma.wait()

        total = local_stats[:, 0:1] + remote_stats[:, 0:1]
        total_sq = local_stats[:, 1:2] + remote_stats[:, 1:2]
        mean = total / n_global
        var = total_sq / n_global - mean * mean
        inv = lax.rsqrt(var + EPS)
        out_ref[:, :] = g_ref[0:1, :] * ((xv - mean) * inv) + b_ref[0:1, :]

    return pl.pallas_call(
        body,
        out_shape=jax.ShapeDtypeStruct((m, n), x.dtype),
        in_specs=[
            pl.BlockSpec(memory_space=pltpu.VMEM),
            pl.BlockSpec(memory_space=pltpu.VMEM),
            pl.BlockSpec(memory_space=pltpu.VMEM),
        ],
        out_specs=pl.BlockSpec(memory_space=pltpu.VMEM),
        scratch_shapes=[
            pltpu.VMEM((m, 2), jnp.float32),
            pltpu.VMEM((m, 2), jnp.float32),
            pltpu.SemaphoreType.DMA,
            pltpu.SemaphoreType.DMA,
        ],
        compiler_params=pltpu.CompilerParams(
            collective_id=0,
            vmem_limit_bytes=64 * 1024 * 1024,
        ),
    )(x, gamma2, beta2)


# device time: 43627 ns/iter; 1.2894x vs baseline; 1.2894x over previous
import jax
import jax.numpy as jnp
from jax import lax
from jax.experimental import pallas as pl
from jax.experimental.pallas import tpu as pltpu

EPS = 1e-5
NBLK = 8
NSLOT = 4


def kernel(x, gamma, beta):
    m, n = x.shape
    bm = m // NBLK
    n_global = 2.0 * n

    gamma2 = gamma.reshape(1, n)
    beta2 = beta.reshape(1, n)

    def body(x_hbm, g_ref, b_ref, out_hbm,
             xbuf, obuf, local_stats, remote_stats,
             in_sems, out_sems, send_sems, recv_sems):
        my_x = lax.axis_index("x")
        my_y = lax.axis_index("y")
        peer = (my_x, 1 - my_y)

        barrier_sem = pltpu.get_barrier_semaphore()
        pl.semaphore_signal(
            barrier_sem, inc=1,
            device_id=peer, device_id_type=pl.DeviceIdType.MESH,
        )

        in_copies = []
        for i in range(NBLK):
            cp = pltpu.make_async_copy(
                x_hbm.at[pl.ds(i * bm, bm), :],
                xbuf.at[pl.ds(i * bm, bm), :],
                in_sems.at[i],
            )
            cp.start()
            in_copies.append(cp)

        pl.semaphore_wait(barrier_sem, 1)

        rdmas = []
        for i in range(NBLK):
            in_copies[i].wait()
            blk = xbuf[pl.ds(i * bm, bm), :]
            local_stats[pl.ds(i * bm, bm), 0:1] = jnp.sum(
                blk, axis=1, keepdims=True)
            local_stats[pl.ds(i * bm, bm), 1:2] = jnp.sum(
                blk * blk, axis=1, keepdims=True)
            rdma = pltpu.make_async_remote_copy(
                src_ref=local_stats.at[pl.ds(i * bm, bm), :],
                dst_ref=remote_stats.at[pl.ds(i * bm, bm), :],
                send_sem=send_sems.at[i],
                recv_sem=recv_sems.at[i],
                device_id=peer,
                device_id_type=pl.DeviceIdType.MESH,
            )
            rdma.start()
            rdmas.append(rdma)

        out_copies = [None] * NBLK
        for i in range(NBLK):
            rdmas[i].wait_recv()
            if i >= NSLOT:
                out_copies[i - NSLOT].wait()
            slot = i % NSLOT
            total = (local_stats[pl.ds(i * bm, bm), 0:1]
                     + remote_stats[pl.ds(i * bm, bm), 0:1])
            total_sq = (local_stats[pl.ds(i * bm, bm), 1:2]
                        + remote_stats[pl.ds(i * bm, bm), 1:2])
            mean = total / n_global
            var = total_sq / n_global - mean * mean
            inv = lax.rsqrt(var + EPS)
            blk = xbuf[pl.ds(i * bm, bm), :]
            obuf[slot] = g_ref[0:1, :] * ((blk - mean) * inv) + b_ref[0:1, :]
            cp = pltpu.make_async_copy(
                obuf.at[slot],
                out_hbm.at[pl.ds(i * bm, bm), :],
                out_sems.at[i],
            )
            cp.start()
            out_copies[i] = cp

        for i in range(NBLK - NSLOT, NBLK):
            out_copies[i].wait()
        for i in range(NBLK):
            rdmas[i].wait_send()

    return pl.pallas_call(
        body,
        out_shape=jax.ShapeDtypeStruct((m, n), x.dtype),
        in_specs=[
            pl.BlockSpec(memory_space=pl.ANY),
            pl.BlockSpec(memory_space=pltpu.VMEM),
            pl.BlockSpec(memory_space=pltpu.VMEM),
        ],
        out_specs=pl.BlockSpec(memory_space=pl.ANY),
        scratch_shapes=[
            pltpu.VMEM((m, n), jnp.float32),
            pltpu.VMEM((NSLOT, bm, n), jnp.float32),
            pltpu.VMEM((m, 2), jnp.float32),
            pltpu.VMEM((m, 2), jnp.float32),
            pltpu.SemaphoreType.DMA((NBLK,)),
            pltpu.SemaphoreType.DMA((NBLK,)),
            pltpu.SemaphoreType.DMA((NBLK,)),
            pltpu.SemaphoreType.DMA((NBLK,)),
        ],
        compiler_params=pltpu.CompilerParams(
            collective_id=0,
            vmem_limit_bytes=64 * 1024 * 1024,
        ),
    )(x, gamma2, beta2)


# device time: 29812 ns/iter; 1.8869x vs baseline; 1.4634x over previous
import jax
import jax.numpy as jnp
from jax import lax
from jax.experimental import pallas as pl
from jax.experimental.pallas import tpu as pltpu

EPS = 1e-5
NBLK = 8
NSLOT = 4
LAG = 2


def kernel(x, gamma, beta):
    m, n = x.shape
    bm = m // NBLK
    sub = bm // 128
    n_global = 2.0 * n

    gamma2 = gamma.reshape(1, n)
    beta2 = beta.reshape(1, n)

    def body(x_hbm, g_ref, b_ref, out_hbm,
             xbuf, obuf, local_stats, remote_stats,
             in_sems, out_sems, send_sems, recv_sems):
        my_x = lax.axis_index("x")
        my_y = lax.axis_index("y")
        peer = (my_x, 1 - my_y)

        r_idx = lax.broadcasted_iota(jnp.int32, (bm, 128), 0)
        b_idx = lax.broadcasted_iota(jnp.int32, (bm, 128), 1)
        unpack_lane = (r_idx % 128 == b_idx).astype(jnp.float32)
        ra_idx = lax.broadcasted_iota(jnp.int32, (bm, sub), 0)
        a_idx = lax.broadcasted_iota(jnp.int32, (bm, sub), 1)
        unpack_sub = (ra_idx // 128 == a_idx).astype(jnp.float32)

        def unpack(packed):
            rows = lax.dot_general(
                unpack_lane, packed,
                (((1,), (1,)), ((), ())),
                preferred_element_type=jnp.float32,
                precision=lax.Precision.HIGHEST,
            )
            return jnp.sum(rows * unpack_sub, axis=1, keepdims=True)

        barrier_sem = pltpu.get_barrier_semaphore()
        pl.semaphore_signal(
            barrier_sem, inc=1,
            device_id=peer, device_id_type=pl.DeviceIdType.MESH,
        )

        in_copies = []
        for i in range(NBLK):
            cp = pltpu.make_async_copy(
                x_hbm.at[pl.ds(i * bm, bm), :],
                xbuf.at[pl.ds(i * bm, bm), :],
                in_sems.at[i],
            )
            cp.start()
            in_copies.append(cp)

        pl.semaphore_wait(barrier_sem, 1)

        rdmas = [None] * NBLK
        out_copies = [None] * NBLK
        for i in range(NBLK + LAG):
            if i < NBLK:
                in_copies[i].wait()
                blk = xbuf[pl.ds(i * bm, bm), :]
                s = jnp.sum(blk, axis=1, keepdims=True).reshape(sub, 128)
                sq = jnp.sum(blk * blk, axis=1, keepdims=True).reshape(sub, 128)
                base = i * 2 * sub
                local_stats[pl.ds(base, sub), :] = s
                local_stats[pl.ds(base + sub, sub), :] = sq
                rdma = pltpu.make_async_remote_copy(
                    src_ref=local_stats.at[pl.ds(base, 2 * sub), :],
                    dst_ref=remote_stats.at[pl.ds(base, 2 * sub), :],
                    send_sem=send_sems.at[i],
                    recv_sem=recv_sems.at[i],
                    device_id=peer,
                    device_id_type=pl.DeviceIdType.MESH,
                )
                rdma.start()
                rdmas[i] = rdma
            if i >= LAG:
                j = i - LAG
                rdmas[j].wait_recv()
                if j >= NSLOT:
                    out_copies[j - NSLOT].wait()
                slot = j % NSLOT
                base = j * 2 * sub
                total = (local_stats[pl.ds(base, sub), :]
                         + remote_stats[pl.ds(base, sub), :])
                total_sq = (local_stats[pl.ds(base + sub, sub), :]
                            + remote_stats[pl.ds(base + sub, sub), :])
                mean = total / n_global
                var = total_sq / n_global - mean * mean
                inv = lax.rsqrt(var + EPS)
                mean_r = unpack(mean)
                inv_r = unpack(inv)
                blk = xbuf[pl.ds(j * bm, bm), :]
                obuf[slot] = (g_ref[0:1, :] * ((blk - mean_r) * inv_r)
                              + b_ref[0:1, :])
                cp = pltpu.make_async_copy(
                    obuf.at[slot],
                    out_hbm.at[pl.ds(j * bm, bm), :],
                    out_sems.at[j],
                )
                cp.start()
                out_copies[j] = cp

        for i in range(NBLK - NSLOT, NBLK):
            out_copies[i].wait()
        for i in range(NBLK):
            rdmas[i].wait_send()

    return pl.pallas_call(
        body,
        out_shape=jax.ShapeDtypeStruct((m, n), x.dtype),
        in_specs=[
            pl.BlockSpec(memory_space=pl.ANY),
            pl.BlockSpec(memory_space=pltpu.VMEM),
            pl.BlockSpec(memory_space=pltpu.VMEM),
        ],
        out_specs=pl.BlockSpec(memory_space=pl.ANY),
        scratch_shapes=[
            pltpu.VMEM((m, n), jnp.float32),
            pltpu.VMEM((NSLOT, bm, n), jnp.float32),
            pltpu.VMEM((NBLK * 2 * (m // NBLK // 128), 128), jnp.float32),
            pltpu.VMEM((NBLK * 2 * (m // NBLK // 128), 128), jnp.float32),
            pltpu.SemaphoreType.DMA((NBLK,)),
            pltpu.SemaphoreType.DMA((NBLK,)),
            pltpu.SemaphoreType.DMA((NBLK,)),
            pltpu.SemaphoreType.DMA((NBLK,)),
        ],
        compiler_params=pltpu.CompilerParams(
            collective_id=0,
            vmem_limit_bytes=64 * 1024 * 1024,
        ),
    )(x, gamma2, beta2)


# device time: 22336 ns/iter; 2.5184x vs baseline; 1.3347x over previous
import jax
import jax.numpy as jnp
from jax import lax
from jax.experimental import pallas as pl
from jax.experimental.pallas import tpu as pltpu

EPS = 1e-5
NBLK = 8
NSLOT = 4


def _stats_exchange(x):
    m, n = x.shape
    bm = m // NBLK
    sub = bm // 128

    def body(x_hbm, out_ref, xbuf, local_stats, remote_stats,
             in_sems, send_sems, recv_sems):
        my_x = lax.axis_index("x")
        my_y = lax.axis_index("y")
        peer = (my_x, 1 - my_y)

        barrier_sem = pltpu.get_barrier_semaphore()
        pl.semaphore_signal(
            barrier_sem, inc=1,
            device_id=peer, device_id_type=pl.DeviceIdType.MESH,
        )

        in_copies = []
        for i in range(NBLK):
            cp = pltpu.make_async_copy(
                x_hbm.at[pl.ds(i * bm, bm), :],
                xbuf.at[pl.ds(i * bm, bm), :],
                in_sems.at[i],
            )
            cp.start()
            in_copies.append(cp)

        pl.semaphore_wait(barrier_sem, 1)

        rdmas = []
        for i in range(NBLK):
            in_copies[i].wait()
            blk = xbuf[pl.ds(i * bm, bm), :]
            s = jnp.sum(blk, axis=1, keepdims=True).reshape(sub, 128)
            sq = jnp.sum(blk * blk, axis=1, keepdims=True).reshape(sub, 128)
            base = i * 2 * sub
            local_stats[pl.ds(base, sub), :] = s
            local_stats[pl.ds(base + sub, sub), :] = sq
            rdma = pltpu.make_async_remote_copy(
                src_ref=local_stats.at[pl.ds(base, 2 * sub), :],
                dst_ref=remote_stats.at[pl.ds(base, 2 * sub), :],
                send_sem=send_sems.at[i],
                recv_sem=recv_sems.at[i],
                device_id=peer,
                device_id_type=pl.DeviceIdType.MESH,
            )
            rdma.start()
            rdmas.append(rdma)

        for r in rdmas:
            r.wait_recv()
        out_ref[:, :] = local_stats[:, :] + remote_stats[:, :]
        for r in rdmas:
            r.wait_send()

    return pl.pallas_call(
        body,
        out_shape=jax.ShapeDtypeStruct((NBLK * 2 * sub, 128), jnp.float32),
        in_specs=[pl.BlockSpec(memory_space=pl.ANY)],
        out_specs=pl.BlockSpec(memory_space=pltpu.VMEM),
        scratch_shapes=[
            pltpu.VMEM((m, n), jnp.float32),
            pltpu.VMEM((NBLK * 2 * sub, 128), jnp.float32),
            pltpu.VMEM((NBLK * 2 * sub, 128), jnp.float32),
            pltpu.SemaphoreType.DMA((NBLK,)),
            pltpu.SemaphoreType.DMA((NBLK,)),
            pltpu.SemaphoreType.DMA((NBLK,)),
        ],
        compiler_params=pltpu.CompilerParams(
            collective_id=0,
            vmem_limit_bytes=64 * 1024 * 1024,
        ),
    )(x)


def _normalize(x, stats, gamma2, beta2):
    m, n = x.shape
    bm = m // NBLK
    sub = bm // 128
    n_global = 2.0 * n

    def body(x_hbm, stats_ref, g_ref, b_ref, out_hbm,
             xbuf, obuf, in_sems, out_sems):
        r_idx = lax.broadcasted_iota(jnp.int32, (bm, 128), 0)
        b_idx = lax.broadcasted_iota(jnp.int32, (bm, 128), 1)
        unpack_lane = (r_idx % 128 == b_idx).astype(jnp.float32)
        ra_idx = lax.broadcasted_iota(jnp.int32, (bm, sub), 0)
        a_idx = lax.broadcasted_iota(jnp.int32, (bm, sub), 1)
        unpack_sub = (ra_idx // 128 == a_idx).astype(jnp.float32)

        def unpack(packed):
            rows = lax.dot_general(
                unpack_lane, packed,
                (((1,), (1,)), ((), ())),
                preferred_element_type=jnp.float32,
                precision=lax.Precision.HIGHEST,
            )
            return jnp.sum(rows * unpack_sub, axis=1, keepdims=True)

        in_copies = []
        for i in range(NBLK):
            cp = pltpu.make_async_copy(
                x_hbm.at[pl.ds(i * bm, bm), :],
                xbuf.at[pl.ds(i * bm, bm), :],
                in_sems.at[i],
            )
            cp.start()
            in_copies.append(cp)

        out_copies = [None] * NBLK
        for i in range(NBLK):
            in_copies[i].wait()
            if i >= NSLOT:
                out_copies[i - NSLOT].wait()
            slot = i % NSLOT
            base = i * 2 * sub
            total = stats_ref[pl.ds(base, sub), :]
            total_sq = stats_ref[pl.ds(base + sub, sub), :]
            mean = total / n_global
            var = total_sq / n_global - mean * mean
            inv = lax.rsqrt(var + EPS)
            mean_r = unpack(mean)
            inv_r = unpack(inv)
            blk = xbuf[pl.ds(i * bm, bm), :]
            obuf[slot] = (g_ref[0:1, :] * ((blk - mean_r) * inv_r)
                          + b_ref[0:1, :])
            cp = pltpu.make_async_copy(
                obuf.at[slot],
                out_hbm.at[pl.ds(i * bm, bm), :],
                out_sems.at[i],
            )
            cp.start()
            out_copies[i] = cp

        for i in range(NBLK - NSLOT, NBLK):
            out_copies[i].wait()

    return pl.pallas_call(
        body,
        out_shape=jax.ShapeDtypeStruct((m, n), x.dtype),
        in_specs=[
            pl.BlockSpec(memory_space=pl.ANY),
            pl.BlockSpec(memory_space=pltpu.VMEM),
            pl.BlockSpec(memory_space=pltpu.VMEM),
            pl.BlockSpec(memory_space=pltpu.VMEM),
        ],
        out_specs=pl.BlockSpec(memory_space=pl.ANY),
        scratch_shapes=[
            pltpu.VMEM((m, n), jnp.float32),
            pltpu.VMEM((NSLOT, bm, n), jnp.float32),
            pltpu.SemaphoreType.DMA((NBLK,)),
            pltpu.SemaphoreType.DMA((NBLK,)),
        ],
        compiler_params=pltpu.CompilerParams(
            vmem_limit_bytes=64 * 1024 * 1024,
        ),
    )(x, stats, gamma2, beta2)


def kernel(x, gamma, beta):
    m, n = x.shape
    gamma2 = gamma.reshape(1, n)
    beta2 = beta.reshape(1, n)
    stats = _stats_exchange(x)
    return _normalize(x, stats, gamma2, beta2)
